# baseline (speedup 1.0000x reference)
import functools
import jax
import jax.numpy as jnp
from jax import lax
from jax.experimental import pallas as pl
from jax.experimental.pallas import tpu as pltpu
from jax.experimental.pallas import tpu_sc as plsc

_T, _D, _H, _E = 2048, 768, 1536, 8
_EPS = 1e-5
_BS = 128
_NB = (2 * _T) // _BS + _E
_SPAD = _NB * _BS
_CH = 256
_WW = 128

_NC = 2
_NS = 16
_NW = _NC * _NS
_RPW = _T // _NW


def _route_kernel(x_ref, g_ref, bta_ref, rw_ref, rb_ref,
                  xn_ref, pos0_ref, pos1_ref, wp0_ref, wp1_ref, be_ref,
                  bv_ref, cex_ref):
    xx = x_ref[...]
    mu = jnp.mean(xx, axis=1, keepdims=True)
    xc = xx - mu
    var = jnp.mean(xc * xc, axis=1, keepdims=True)
    xn = xc * lax.rsqrt(var + _EPS)
    xn = xn * g_ref[...] + bta_ref[...]
    xn_ref[...] = xn

    logits = jnp.dot(xn, rw_ref[...],
                     preferred_element_type=jnp.float32) + rb_ref[...]
    ids = lax.broadcasted_iota(jnp.int32, (_T, _E), 1)
    m1 = jnp.max(logits, axis=1, keepdims=True)
    i1 = jnp.min(jnp.where(logits == m1, ids, _E), axis=1, keepdims=True)
    l2 = jnp.where(ids == i1, -jnp.inf, logits)
    m2 = jnp.max(l2, axis=1, keepdims=True)
    i2 = jnp.min(jnp.where(l2 == m2, ids, _E), axis=1, keepdims=True)
    e2 = jnp.exp(m2 - m1)
    denom = 1.0 + e2
    w0 = 1.0 / denom
    w1 = e2 / denom
    col0 = lax.broadcasted_iota(jnp.int32, (1, _WW), 1) == 0
    wp0_ref[...] = jnp.where(col0, w0, 0.0)
    wp1_ref[...] = jnp.where(col0, w1, 0.0)

    hit1 = ids == i1
    hit2 = ids == i2
    m_bf = (jnp.where(hit1, 1.0, 0.0)
            + jnp.where(hit2, 1.0, 0.0)).astype(jnp.bfloat16)

    rr = lax.broadcasted_iota(jnp.int32, (_CH, _CH), 0)
    cc = lax.broadcasted_iota(jnp.int32, (_CH, _CH), 1)
    tri = jnp.where(cc < rr, 1.0, 0.0).astype(jnp.bfloat16)
    carry = jnp.zeros((1, _E), jnp.float32)
    for c in range(_T // _CH):
        mc = lax.slice(m_bf, (c * _CH, 0), ((c + 1) * _CH, _E))
        cex_ref[pl.ds(c * _CH, _CH), :] = (
            jnp.dot(tri, mc, preferred_element_type=jnp.float32) + carry)
        carry = carry + jnp.sum(mc.astype(jnp.float32), axis=0, keepdims=True)

    cnt_row = carry
    nblk_row = jnp.ceil(cnt_row * (1.0 / _BS))
    eu = lax.broadcasted_iota(jnp.int32, (_E, _E), 0)
    ev = lax.broadcasted_iota(jnp.int32, (_E, _E), 1)
    upper = jnp.where(eu < ev, 1.0, 0.0)
    bs_row = jnp.dot(nblk_row, upper,
                     preferred_element_type=jnp.float32)
    nbtot = jnp.sum(nblk_row, axis=1, keepdims=True)
    base_row = bs_row * float(_BS)

    posf = base_row + cex_ref[...]
    p0 = jnp.sum(jnp.where(hit1, posf, 0.0), axis=1, keepdims=True)
    p1 = jnp.sum(jnp.where(hit2, posf, 0.0), axis=1, keepdims=True)
    pos0_ref[...] = p0.astype(jnp.int32).reshape(_T)
    pos1_ref[...] = p1.astype(jnp.int32).reshape(_T)

    ones_t1 = jnp.zeros((_T, 1), jnp.float32) + 1.0
    cnt_col = lax.dot_general(m_bf.astype(jnp.float32), ones_t1,
                              (((0,), (0,)), ((), ())))
    nblk_col = jnp.ceil(cnt_col * (1.0 / _BS))
    bs_col = lax.dot_general(upper, nblk_col,
                             (((0,), (0,)), ((), ())))
    ib = lax.broadcasted_iota(jnp.int32, (_E, _NB), 1).astype(jnp.float32)
    be_raw = jnp.sum(jnp.where(bs_col <= ib, 1.0, 0.0),
                     axis=0, keepdims=True) - 1.0
    ibr = lax.broadcasted_iota(jnp.int32, (1, _NB), 1).astype(jnp.float32)
    bv = jnp.where(ibr < nbtot, 1, 0)
    belast = jnp.sum(jnp.where(bs_col <= (nbtot - 1.0), 1.0, 0.0),
                     axis=0, keepdims=True) - 1.0
    be_ref[...] = jnp.where(bv == 1, be_raw, belast).astype(jnp.int32).reshape(_NB)
    bv_ref[...] = bv.astype(jnp.int32).reshape(_NB)


def _route(x, g2, bta2, rw, rb2):
    full = lambda shape: pl.BlockSpec(shape, lambda: (0,) * len(shape))
    return pl.pallas_call(
        _route_kernel,
        grid=(),
        in_specs=[full((_T, _D)), full((1, _D)), full((1, _D)),
                  full((_D, _E)), full((1, _E))],
        out_specs=[full((_T, _D)), full((_T,)), full((_T,)),
                   full((_T, _WW)), full((_T, _WW)), full((_NB,)),
                   full((_NB,))],
        out_shape=[
            jax.ShapeDtypeStruct((_T, _D), jnp.float32),
            jax.ShapeDtypeStruct((_T,), jnp.int32),
            jax.ShapeDtypeStruct((_T,), jnp.int32),
            jax.ShapeDtypeStruct((_T, _WW), jnp.float32),
            jax.ShapeDtypeStruct((_T, _WW), jnp.float32),
            jax.ShapeDtypeStruct((_NB,), jnp.int32),
            jax.ShapeDtypeStruct((_NB,), jnp.int32),
        ],
        scratch_shapes=[pltpu.VMEM((_T, _E), jnp.float32)],
    )(x, g2, bta2, rw, rb2)


def _scatter_rows(xn, wp0, wp1, pos0, pos1):
    mesh = plsc.VectorSubcoreMesh(core_axis_name="c", subcore_axis_name="s")

    @functools.partial(
        pl.kernel, mesh=mesh,
        out_type=[jax.ShapeDtypeStruct((_SPAD, _D), jnp.float32),
                  jax.ShapeDtypeStruct((_SPAD, _WW), jnp.float32)],
        scratch_types=[pltpu.VMEM((_RPW,), jnp.int32),
                       pltpu.VMEM((_RPW, _D), jnp.float32),
                       pltpu.VMEM((_RPW, _WW), jnp.float32),
                       pltpu.SemaphoreType.DMA],
    )
    def k(xn_hbm, wp0_hbm, wp1_hbm, p0_hbm, p1_hbm, xs_hbm, ws_hbm,
          idx_v, rows_v, wrow_v, sem):
        wid = lax.axis_index("s") * _NC + lax.axis_index("c")
        base = wid * _RPW
        pltpu.sync_copy(xn_hbm.at[pl.ds(base, _RPW)], rows_v)
        pltpu.sync_copy(p0_hbm.at[pl.ds(base, _RPW)], idx_v)
        pltpu.sync_copy(wp0_hbm.at[pl.ds(base, _RPW)], wrow_v)
        pltpu.async_copy(rows_v, xs_hbm.at[idx_v], sem).wait()
        pltpu.async_copy(wrow_v, ws_hbm.at[idx_v], sem).wait()
        pltpu.sync_copy(p1_hbm.at[pl.ds(base, _RPW)], idx_v)
        pltpu.sync_copy(wp1_hbm.at[pl.ds(base, _RPW)], wrow_v)
        pltpu.async_copy(rows_v, xs_hbm.at[idx_v], sem).wait()
        pltpu.async_copy(wrow_v, ws_hbm.at[idx_v], sem).wait()

    return k(xn, wp0, wp1, pos0, pos1)


def _ffn_kernel(be_ref, bv_ref, xs_ref, ws_ref, w1_ref, b1_ref, w2_ref,
                b2_ref, ys_ref, w1s_ref, w2s_ref):
    i = pl.program_id(0)

    @pl.when(bv_ref[i] == 1)
    def _():
        prev = be_ref[jnp.maximum(i - 1, 0)]

        @pl.when((i == 0) | (be_ref[i] != prev))
        def _():
            w1s_ref[...] = w1_ref[0].astype(jnp.bfloat16)
            w2s_ref[...] = w2_ref[0].astype(jnp.bfloat16)

        e = be_ref[i]
        xb = xs_ref[...].astype(jnp.bfloat16)
        hh = _H // 2
        yb = b2_ref[pl.ds(e, 1), :] * 1.0
        b1e = b1_ref[pl.ds(e, 1), :]
        for p in range(2):
            w1h = w1s_ref[:, pl.ds(p * hh, hh)]
            b1h = lax.slice(b1e, (0, p * hh), (1, (p + 1) * hh))
            h = jnp.dot(xb, w1h, preferred_element_type=jnp.float32) + b1h
            h = 0.5 * h * (1.0 + lax.erf(h * 0.7071067811865476))
            yb = yb + jnp.dot(h.astype(jnp.bfloat16),
                              w2s_ref[pl.ds(p * hh, hh), :],
                              preferred_element_type=jnp.float32)
        ys_ref[...] = yb * ws_ref[:, 0:1]


def _ffn(be, bv, xs, ws, W1, b1, W2, b2):
    clamp = lambda i, be, bv: jnp.where(bv[i] == 1, i, 0)
    grid_spec = pltpu.PrefetchScalarGridSpec(
        num_scalar_prefetch=2,
        grid=(_NB,),
        in_specs=[
            pl.BlockSpec((_BS, _D), lambda i, be, bv: (clamp(i, be, bv), 0)),
            pl.BlockSpec((_BS, _WW), lambda i, be, bv: (clamp(i, be, bv), 0)),
            pl.BlockSpec((1, _D, _H), lambda i, be, bv: (be[i], 0, 0)),
            pl.BlockSpec((_E, _H), lambda i, be, bv: (0, 0)),
            pl.BlockSpec((1, _H, _D), lambda i, be, bv: (be[i], 0, 0)),
            pl.BlockSpec((_E, _D), lambda i, be, bv: (0, 0)),
        ],
        out_specs=pl.BlockSpec((_BS, _D), lambda i, be, bv: (i, 0)),
        scratch_shapes=[pltpu.VMEM((_D, _H), jnp.bfloat16),
                        pltpu.VMEM((_H, _D), jnp.bfloat16)],
    )
    return pl.pallas_call(
        _ffn_kernel,
        grid_spec=grid_spec,
        out_shape=jax.ShapeDtypeStruct((_SPAD, _D), jnp.float32),
        compiler_params=pltpu.CompilerParams(
            dimension_semantics=("arbitrary",)),
    )(be, bv, xs, ws, W1, b1, W2, b2)


def _gather_combine(ys, pos0, pos1):
    mesh = plsc.VectorSubcoreMesh(core_axis_name="c", subcore_axis_name="s")

    @functools.partial(
        pl.kernel, mesh=mesh,
        out_type=jax.ShapeDtypeStruct((_T, _D), jnp.float32),
        scratch_types=[pltpu.VMEM((_RPW,), jnp.int32),
                       pltpu.VMEM((_RPW,), jnp.int32),
                       pltpu.VMEM((_RPW // 2, _D), jnp.float32),
                       pltpu.VMEM((_RPW // 2, _D), jnp.float32),
                       pltpu.VMEM((_RPW // 2, _D), jnp.float32),
                       pltpu.VMEM((_RPW // 2, _D), jnp.float32),
                       pltpu.SemaphoreType.DMA,
                       pltpu.SemaphoreType.DMA,
                       pltpu.SemaphoreType.DMA,
                       pltpu.SemaphoreType.DMA],
    )
    def k(ys_hbm, p0_hbm, p1_hbm, y_hbm, idx0_v, idx1_v, a0_v, b0_v,
          a1_v, b1_v, s0, s1, s2, s3):
        wid = lax.axis_index("s") * _NC + lax.axis_index("c")
        base = wid * _RPW
        hp = _RPW // 2
        pltpu.sync_copy(p0_hbm.at[pl.ds(base, _RPW)], idx0_v)
        pltpu.sync_copy(p1_hbm.at[pl.ds(base, _RPW)], idx1_v)
        c0 = pltpu.async_copy(ys_hbm.at[idx0_v.at[pl.ds(0, hp)]], a0_v, s0)
        c1 = pltpu.async_copy(ys_hbm.at[idx1_v.at[pl.ds(0, hp)]], b0_v, s1)
        c2 = pltpu.async_copy(ys_hbm.at[idx0_v.at[pl.ds(hp, hp)]], a1_v, s2)
        c3 = pltpu.async_copy(ys_hbm.at[idx1_v.at[pl.ds(hp, hp)]], b1_v, s3)

        def add_store(a_v, b_v, off):
            @pl.loop(0, hp)
            def _(r):
                @pl.loop(0, _D, step=16)
                def _(cc):
                    slc = (pl.ds(r, 1), pl.ds(cc, 16))
                    a_v.at[*slc][...] = a_v.at[*slc][...] + b_v.at[*slc][...]
            pltpu.sync_copy(a_v, y_hbm.at[pl.ds(base + off, hp)])

        c0.wait()
        c1.wait()
        add_store(a0_v, b0_v, 0)
        c2.wait()
        c3.wait()
        add_store(a1_v, b1_v, hp)

    return k(ys, pos0, pos1)


def kernel(x, ln_gamma, ln_beta, router_W, router_b, W1, b1, W2, b2):
    g2 = ln_gamma.reshape(1, _D)
    bta2 = ln_beta.reshape(1, _D)
    rb2 = router_b.reshape(1, _E)

    xn, pos0, pos1, wp0, wp1, be2, bv2 = _route(x, g2, bta2, router_W, rb2)
    xs, ws = _scatter_rows(xn, wp0, wp1, pos0, pos1)
    ys = _ffn(be2, bv2, xs, ws, W1, b1, W2, b2)
    return _gather_combine(ys, pos0, pos1)


# speedup vs baseline: 1.1185x; 1.1185x over previous
import functools
import jax
import jax.numpy as jnp
from jax import lax
from jax.experimental import pallas as pl
from jax.experimental.pallas import tpu as pltpu
from jax.experimental.pallas import tpu_sc as plsc

_T, _D, _H, _E = 2048, 768, 1536, 8
_EPS = 1e-5
_BS = 512
_NB = (2 * _T) // _BS + _E
_SPAD = _NB * _BS
_CH = 256
_WW = 128

_NC = 2
_NS = 16
_NW = _NC * _NS
_RPW = _T // _NW


def _route_kernel(x_ref, g_ref, bta_ref, rw_ref, rb_ref,
                  xn_ref, pos0_ref, pos1_ref, wp0_ref, wp1_ref, be_ref,
                  bv_ref, cex_ref):
    xx = x_ref[...]
    mu = jnp.mean(xx, axis=1, keepdims=True)
    xc = xx - mu
    var = jnp.mean(xc * xc, axis=1, keepdims=True)
    xn = xc * lax.rsqrt(var + _EPS)
    xn = xn * g_ref[...] + bta_ref[...]
    xn_ref[...] = xn

    logits = jnp.dot(xn, rw_ref[...],
                     preferred_element_type=jnp.float32) + rb_ref[...]
    ids = lax.broadcasted_iota(jnp.int32, (_T, _E), 1)
    m1 = jnp.max(logits, axis=1, keepdims=True)
    i1 = jnp.min(jnp.where(logits == m1, ids, _E), axis=1, keepdims=True)
    l2 = jnp.where(ids == i1, -jnp.inf, logits)
    m2 = jnp.max(l2, axis=1, keepdims=True)
    i2 = jnp.min(jnp.where(l2 == m2, ids, _E), axis=1, keepdims=True)
    e2 = jnp.exp(m2 - m1)
    denom = 1.0 + e2
    w0 = 1.0 / denom
    w1 = e2 / denom
    col0 = lax.broadcasted_iota(jnp.int32, (1, _WW), 1) == 0
    wp0_ref[...] = jnp.where(col0, w0, 0.0)
    wp1_ref[...] = jnp.where(col0, w1, 0.0)

    hit1 = ids == i1
    hit2 = ids == i2
    m_bf = (jnp.where(hit1, 1.0, 0.0)
            + jnp.where(hit2, 1.0, 0.0)).astype(jnp.bfloat16)

    rr = lax.broadcasted_iota(jnp.int32, (_CH, _CH), 0)
    cc = lax.broadcasted_iota(jnp.int32, (_CH, _CH), 1)
    tri = jnp.where(cc < rr, 1.0, 0.0).astype(jnp.bfloat16)
    carry = jnp.zeros((1, _E), jnp.float32)
    for c in range(_T // _CH):
        mc = lax.slice(m_bf, (c * _CH, 0), ((c + 1) * _CH, _E))
        cex_ref[pl.ds(c * _CH, _CH), :] = (
            jnp.dot(tri, mc, preferred_element_type=jnp.float32) + carry)
        carry = carry + jnp.sum(mc.astype(jnp.float32), axis=0, keepdims=True)

    cnt_row = carry
    nblk_row = jnp.ceil(cnt_row * (1.0 / _BS))
    eu = lax.broadcasted_iota(jnp.int32, (_E, _E), 0)
    ev = lax.broadcasted_iota(jnp.int32, (_E, _E), 1)
    upper = jnp.where(eu < ev, 1.0, 0.0)
    bs_row = jnp.dot(nblk_row, upper,
                     preferred_element_type=jnp.float32)
    nbtot = jnp.sum(nblk_row, axis=1, keepdims=True)
    base_row = bs_row * float(_BS)

    posf = base_row + cex_ref[...]
    p0 = jnp.sum(jnp.where(hit1, posf, 0.0), axis=1, keepdims=True)
    p1 = jnp.sum(jnp.where(hit2, posf, 0.0), axis=1, keepdims=True)
    pos0_ref[...] = p0.astype(jnp.int32).reshape(_T)
    pos1_ref[...] = p1.astype(jnp.int32).reshape(_T)

    ones_t1 = jnp.zeros((_T, 1), jnp.float32) + 1.0
    cnt_col = lax.dot_general(m_bf.astype(jnp.float32), ones_t1,
                              (((0,), (0,)), ((), ())))
    nblk_col = jnp.ceil(cnt_col * (1.0 / _BS))
    bs_col = lax.dot_general(upper, nblk_col,
                             (((0,), (0,)), ((), ())))
    ib = lax.broadcasted_iota(jnp.int32, (_E, _NB), 1).astype(jnp.float32)
    be_raw = jnp.sum(jnp.where(bs_col <= ib, 1.0, 0.0),
                     axis=0, keepdims=True) - 1.0
    ibr = lax.broadcasted_iota(jnp.int32, (1, _NB), 1).astype(jnp.float32)
    bv = jnp.where(ibr < nbtot, 1, 0)
    belast = jnp.sum(jnp.where(bs_col <= (nbtot - 1.0), 1.0, 0.0),
                     axis=0, keepdims=True) - 1.0
    be_ref[...] = jnp.where(bv == 1, be_raw, belast).astype(jnp.int32).reshape(_NB)
    bv_ref[...] = bv.astype(jnp.int32).reshape(_NB)


def _route(x, g2, bta2, rw, rb2):
    full = lambda shape: pl.BlockSpec(shape, lambda: (0,) * len(shape))
    return pl.pallas_call(
        _route_kernel,
        grid=(),
        in_specs=[full((_T, _D)), full((1, _D)), full((1, _D)),
                  full((_D, _E)), full((1, _E))],
        out_specs=[full((_T, _D)), full((_T,)), full((_T,)),
                   full((_T, _WW)), full((_T, _WW)), full((_NB,)),
                   full((_NB,))],
        out_shape=[
            jax.ShapeDtypeStruct((_T, _D), jnp.float32),
            jax.ShapeDtypeStruct((_T,), jnp.int32),
            jax.ShapeDtypeStruct((_T,), jnp.int32),
            jax.ShapeDtypeStruct((_T, _WW), jnp.float32),
            jax.ShapeDtypeStruct((_T, _WW), jnp.float32),
            jax.ShapeDtypeStruct((_NB,), jnp.int32),
            jax.ShapeDtypeStruct((_NB,), jnp.int32),
        ],
        scratch_shapes=[pltpu.VMEM((_T, _E), jnp.float32)],
    )(x, g2, bta2, rw, rb2)


def _scatter_rows(xn, wp0, wp1, pos0, pos1):
    mesh = plsc.VectorSubcoreMesh(core_axis_name="c", subcore_axis_name="s")

    @functools.partial(
        pl.kernel, mesh=mesh,
        out_type=[jax.ShapeDtypeStruct((_SPAD, _D), jnp.float32),
                  jax.ShapeDtypeStruct((_SPAD, _WW), jnp.float32)],
        scratch_types=[pltpu.VMEM((_RPW,), jnp.int32),
                       pltpu.VMEM((_RPW, _D), jnp.float32),
                       pltpu.VMEM((_RPW, _WW), jnp.float32),
                       pltpu.SemaphoreType.DMA],
    )
    def k(xn_hbm, wp0_hbm, wp1_hbm, p0_hbm, p1_hbm, xs_hbm, ws_hbm,
          idx_v, rows_v, wrow_v, sem):
        wid = lax.axis_index("s") * _NC + lax.axis_index("c")
        base = wid * _RPW
        pltpu.sync_copy(xn_hbm.at[pl.ds(base, _RPW)], rows_v)
        pltpu.sync_copy(p0_hbm.at[pl.ds(base, _RPW)], idx_v)
        pltpu.sync_copy(wp0_hbm.at[pl.ds(base, _RPW)], wrow_v)
        pltpu.async_copy(rows_v, xs_hbm.at[idx_v], sem).wait()
        pltpu.async_copy(wrow_v, ws_hbm.at[idx_v], sem).wait()
        pltpu.sync_copy(p1_hbm.at[pl.ds(base, _RPW)], idx_v)
        pltpu.sync_copy(wp1_hbm.at[pl.ds(base, _RPW)], wrow_v)
        pltpu.async_copy(rows_v, xs_hbm.at[idx_v], sem).wait()
        pltpu.async_copy(wrow_v, ws_hbm.at[idx_v], sem).wait()

    return k(xn, wp0, wp1, pos0, pos1)


def _ffn_kernel(be_ref, bv_ref, xs_ref, ws_ref, w1_ref, b1_ref, w2_ref,
                b2_ref, ys_ref, w1s_ref, w2s_ref):
    i = pl.program_id(0)

    @pl.when(bv_ref[i] == 1)
    def _():
        prev = be_ref[jnp.maximum(i - 1, 0)]

        @pl.when((i == 0) | (be_ref[i] != prev))
        def _():
            w1s_ref[...] = w1_ref[0].astype(jnp.bfloat16)
            w2s_ref[...] = w2_ref[0].astype(jnp.bfloat16)

        e = be_ref[i]
        xb = xs_ref[...].astype(jnp.bfloat16)
        hh = _H // 2
        yb = b2_ref[pl.ds(e, 1), :] * 1.0
        b1e = b1_ref[pl.ds(e, 1), :]
        for p in range(2):
            w1h = w1s_ref[:, pl.ds(p * hh, hh)]
            b1h = lax.slice(b1e, (0, p * hh), (1, (p + 1) * hh))
            h = jnp.dot(xb, w1h, preferred_element_type=jnp.float32) + b1h
            h = 0.5 * h * (1.0 + lax.erf(h * 0.7071067811865476))
            yb = yb + jnp.dot(h.astype(jnp.bfloat16),
                              w2s_ref[pl.ds(p * hh, hh), :],
                              preferred_element_type=jnp.float32)
        ys_ref[...] = yb * ws_ref[:, 0:1]


def _ffn(be, bv, xs, ws, W1, b1, W2, b2):
    clamp = lambda i, be, bv: jnp.where(bv[i] == 1, i, 0)
    grid_spec = pltpu.PrefetchScalarGridSpec(
        num_scalar_prefetch=2,
        grid=(_NB,),
        in_specs=[
            pl.BlockSpec((_BS, _D), lambda i, be, bv: (clamp(i, be, bv), 0)),
            pl.BlockSpec((_BS, _WW), lambda i, be, bv: (clamp(i, be, bv), 0)),
            pl.BlockSpec((1, _D, _H), lambda i, be, bv: (be[i], 0, 0)),
            pl.BlockSpec((_E, _H), lambda i, be, bv: (0, 0)),
            pl.BlockSpec((1, _H, _D), lambda i, be, bv: (be[i], 0, 0)),
            pl.BlockSpec((_E, _D), lambda i, be, bv: (0, 0)),
        ],
        out_specs=pl.BlockSpec((_BS, _D), lambda i, be, bv: (i, 0)),
        scratch_shapes=[pltpu.VMEM((_D, _H), jnp.bfloat16),
                        pltpu.VMEM((_H, _D), jnp.bfloat16)],
    )
    return pl.pallas_call(
        _ffn_kernel,
        grid_spec=grid_spec,
        out_shape=jax.ShapeDtypeStruct((_SPAD, _D), jnp.float32),
        compiler_params=pltpu.CompilerParams(
            dimension_semantics=("arbitrary",)),
    )(be, bv, xs, ws, W1, b1, W2, b2)


def _gather_combine(ys, pos0, pos1):
    mesh = plsc.VectorSubcoreMesh(core_axis_name="c", subcore_axis_name="s")

    @functools.partial(
        pl.kernel, mesh=mesh,
        out_type=jax.ShapeDtypeStruct((_T, _D), jnp.float32),
        scratch_types=[pltpu.VMEM((_RPW,), jnp.int32),
                       pltpu.VMEM((_RPW,), jnp.int32),
                       pltpu.VMEM((_RPW // 2, _D), jnp.float32),
                       pltpu.VMEM((_RPW // 2, _D), jnp.float32),
                       pltpu.VMEM((_RPW // 2, _D), jnp.float32),
                       pltpu.VMEM((_RPW // 2, _D), jnp.float32),
                       pltpu.SemaphoreType.DMA,
                       pltpu.SemaphoreType.DMA,
                       pltpu.SemaphoreType.DMA,
                       pltpu.SemaphoreType.DMA],
    )
    def k(ys_hbm, p0_hbm, p1_hbm, y_hbm, idx0_v, idx1_v, a0_v, b0_v,
          a1_v, b1_v, s0, s1, s2, s3):
        wid = lax.axis_index("s") * _NC + lax.axis_index("c")
        base = wid * _RPW
        hp = _RPW // 2
        pltpu.sync_copy(p0_hbm.at[pl.ds(base, _RPW)], idx0_v)
        pltpu.sync_copy(p1_hbm.at[pl.ds(base, _RPW)], idx1_v)
        c0 = pltpu.async_copy(ys_hbm.at[idx0_v.at[pl.ds(0, hp)]], a0_v, s0)
        c1 = pltpu.async_copy(ys_hbm.at[idx1_v.at[pl.ds(0, hp)]], b0_v, s1)
        c2 = pltpu.async_copy(ys_hbm.at[idx0_v.at[pl.ds(hp, hp)]], a1_v, s2)
        c3 = pltpu.async_copy(ys_hbm.at[idx1_v.at[pl.ds(hp, hp)]], b1_v, s3)

        def add_store(a_v, b_v, off):
            @pl.loop(0, hp)
            def _(r):
                @pl.loop(0, _D, step=16)
                def _(cc):
                    slc = (pl.ds(r, 1), pl.ds(cc, 16))
                    a_v.at[*slc][...] = a_v.at[*slc][...] + b_v.at[*slc][...]
            pltpu.sync_copy(a_v, y_hbm.at[pl.ds(base + off, hp)])

        c0.wait()
        c1.wait()
        add_store(a0_v, b0_v, 0)
        c2.wait()
        c3.wait()
        add_store(a1_v, b1_v, hp)

    return k(ys, pos0, pos1)


def kernel(x, ln_gamma, ln_beta, router_W, router_b, W1, b1, W2, b2):
    g2 = ln_gamma.reshape(1, _D)
    bta2 = ln_beta.reshape(1, _D)
    rb2 = router_b.reshape(1, _E)

    xn, pos0, pos1, wp0, wp1, be2, bv2 = _route(x, g2, bta2, router_W, rb2)
    xs, ws = _scatter_rows(xn, wp0, wp1, pos0, pos1)
    ys = _ffn(be2, bv2, xs, ws, W1, b1, W2, b2)
    return _gather_combine(ys, pos0, pos1)


# speedup vs baseline: 1.1419x; 1.0209x over previous
import functools
import jax
import jax.numpy as jnp
from jax import lax
from jax.experimental import pallas as pl
from jax.experimental.pallas import tpu as pltpu
from jax.experimental.pallas import tpu_sc as plsc

_T, _D, _H, _E = 2048, 768, 1536, 8
_EPS = 1e-5
_BS = 512
_NB = (2 * _T) // _BS + _E
_SPAD = _NB * _BS
_CH = 256
_WW = 128

_NC = 2
_NS = 16
_NW = _NC * _NS
_RPW = _T // _NW


def _route_kernel(x_ref, g_ref, bta_ref, rw_ref, rb_ref,
                  xn_ref, pos0_ref, pos1_ref, wp0_ref, wp1_ref, be_ref,
                  bv_ref, cex_ref):
    xx = x_ref[...]
    mu = jnp.mean(xx, axis=1, keepdims=True)
    xc = xx - mu
    var = jnp.mean(xc * xc, axis=1, keepdims=True)
    xn = xc * lax.rsqrt(var + _EPS)
    xn = xn * g_ref[...][None, :] + bta_ref[...][None, :]
    xn_ref[...] = xn

    logits = jnp.dot(xn, rw_ref[...],
                     preferred_element_type=jnp.float32) + rb_ref[...][None, :]
    ids = lax.broadcasted_iota(jnp.int32, (_T, _E), 1)
    m1 = jnp.max(logits, axis=1, keepdims=True)
    i1 = jnp.min(jnp.where(logits == m1, ids, _E), axis=1, keepdims=True)
    l2 = jnp.where(ids == i1, -jnp.inf, logits)
    m2 = jnp.max(l2, axis=1, keepdims=True)
    i2 = jnp.min(jnp.where(l2 == m2, ids, _E), axis=1, keepdims=True)
    e2 = jnp.exp(m2 - m1)
    denom = 1.0 + e2
    w0 = 1.0 / denom
    w1 = e2 / denom
    col0 = lax.broadcasted_iota(jnp.int32, (1, _WW), 1) == 0
    wp0_ref[...] = jnp.where(col0, w0, 0.0)
    wp1_ref[...] = jnp.where(col0, w1, 0.0)

    hit1 = ids == i1
    hit2 = ids == i2
    m_bf = (jnp.where(hit1, 1.0, 0.0)
            + jnp.where(hit2, 1.0, 0.0)).astype(jnp.bfloat16)

    rr = lax.broadcasted_iota(jnp.int32, (_CH, _CH), 0)
    cc = lax.broadcasted_iota(jnp.int32, (_CH, _CH), 1)
    tri = jnp.where(cc < rr, 1.0, 0.0).astype(jnp.bfloat16)
    carry = jnp.zeros((1, _E), jnp.float32)
    for c in range(_T // _CH):
        mc = lax.slice(m_bf, (c * _CH, 0), ((c + 1) * _CH, _E))
        cex_ref[pl.ds(c * _CH, _CH), :] = (
            jnp.dot(tri, mc, preferred_element_type=jnp.float32) + carry)
        carry = carry + jnp.sum(mc.astype(jnp.float32), axis=0, keepdims=True)

    cnt_row = carry
    nblk_row = jnp.ceil(cnt_row * (1.0 / _BS))
    eu = lax.broadcasted_iota(jnp.int32, (_E, _E), 0)
    ev = lax.broadcasted_iota(jnp.int32, (_E, _E), 1)
    upper = jnp.where(eu < ev, 1.0, 0.0)
    bs_row = jnp.dot(nblk_row, upper,
                     preferred_element_type=jnp.float32)
    nbtot = jnp.sum(nblk_row, axis=1, keepdims=True)
    base_row = bs_row * float(_BS)

    posf = base_row + cex_ref[...]
    p0 = jnp.sum(jnp.where(hit1, posf, 0.0), axis=1, keepdims=True)
    p1 = jnp.sum(jnp.where(hit2, posf, 0.0), axis=1, keepdims=True)
    pos0_ref[...] = p0.astype(jnp.int32).reshape(_T)
    pos1_ref[...] = p1.astype(jnp.int32).reshape(_T)

    ones_t1 = jnp.zeros((_T, 1), jnp.float32) + 1.0
    cnt_col = lax.dot_general(m_bf.astype(jnp.float32), ones_t1,
                              (((0,), (0,)), ((), ())))
    nblk_col = jnp.ceil(cnt_col * (1.0 / _BS))
    bs_col = lax.dot_general(upper, nblk_col,
                             (((0,), (0,)), ((), ())))
    ib = lax.broadcasted_iota(jnp.int32, (_E, _NB), 1).astype(jnp.float32)
    be_raw = jnp.sum(jnp.where(bs_col <= ib, 1.0, 0.0),
                     axis=0, keepdims=True) - 1.0
    ibr = lax.broadcasted_iota(jnp.int32, (1, _NB), 1).astype(jnp.float32)
    bv = jnp.where(ibr < nbtot, 1, 0)
    belast = jnp.sum(jnp.where(bs_col <= (nbtot - 1.0), 1.0, 0.0),
                     axis=0, keepdims=True) - 1.0
    be_ref[...] = jnp.where(bv == 1, be_raw, belast).astype(jnp.int32).reshape(_NB)
    bv_ref[...] = bv.astype(jnp.int32).reshape(_NB)


def _route(x, g2, bta2, rw, rb2):
    full = lambda shape: pl.BlockSpec(shape, lambda: (0,) * len(shape))
    return pl.pallas_call(
        _route_kernel,
        grid=(),
        in_specs=[full((_T, _D)), full((_D,)), full((_D,)),
                  full((_D, _E)), full((_E,))],
        out_specs=[full((_T, _D)), full((_T,)), full((_T,)),
                   full((_T, _WW)), full((_T, _WW)), full((_NB,)),
                   full((_NB,))],
        out_shape=[
            jax.ShapeDtypeStruct((_T, _D), jnp.float32),
            jax.ShapeDtypeStruct((_T,), jnp.int32),
            jax.ShapeDtypeStruct((_T,), jnp.int32),
            jax.ShapeDtypeStruct((_T, _WW), jnp.float32),
            jax.ShapeDtypeStruct((_T, _WW), jnp.float32),
            jax.ShapeDtypeStruct((_NB,), jnp.int32),
            jax.ShapeDtypeStruct((_NB,), jnp.int32),
        ],
        scratch_shapes=[pltpu.VMEM((_T, _E), jnp.float32)],
    )(x, g2, bta2, rw, rb2)


def _scatter_rows(xn, wp0, wp1, pos0, pos1):
    mesh = plsc.VectorSubcoreMesh(core_axis_name="c", subcore_axis_name="s")

    @functools.partial(
        pl.kernel, mesh=mesh,
        out_type=[jax.ShapeDtypeStruct((_SPAD, _D), jnp.float32),
                  jax.ShapeDtypeStruct((_SPAD, _WW), jnp.float32)],
        scratch_types=[pltpu.VMEM((_RPW,), jnp.int32),
                       pltpu.VMEM((_RPW, _D), jnp.float32),
                       pltpu.VMEM((_RPW, _WW), jnp.float32),
                       pltpu.SemaphoreType.DMA],
    )
    def k(xn_hbm, wp0_hbm, wp1_hbm, p0_hbm, p1_hbm, xs_hbm, ws_hbm,
          idx_v, rows_v, wrow_v, sem):
        wid = lax.axis_index("s") * _NC + lax.axis_index("c")
        base = wid * _RPW
        pltpu.sync_copy(xn_hbm.at[pl.ds(base, _RPW)], rows_v)
        pltpu.sync_copy(p0_hbm.at[pl.ds(base, _RPW)], idx_v)
        pltpu.sync_copy(wp0_hbm.at[pl.ds(base, _RPW)], wrow_v)
        pltpu.async_copy(rows_v, xs_hbm.at[idx_v], sem).wait()
        pltpu.async_copy(wrow_v, ws_hbm.at[idx_v], sem).wait()
        pltpu.sync_copy(p1_hbm.at[pl.ds(base, _RPW)], idx_v)
        pltpu.sync_copy(wp1_hbm.at[pl.ds(base, _RPW)], wrow_v)
        pltpu.async_copy(rows_v, xs_hbm.at[idx_v], sem).wait()
        pltpu.async_copy(wrow_v, ws_hbm.at[idx_v], sem).wait()

    return k(xn, wp0, wp1, pos0, pos1)


def _ffn_kernel(be_ref, bv_ref, xs_ref, ws_ref, w1_ref, b1_ref, w2_ref,
                b2_ref, ys_ref, w1s_ref, w2s_ref):
    i = pl.program_id(0)

    @pl.when(bv_ref[i] == 1)
    def _():
        prev = be_ref[jnp.maximum(i - 1, 0)]

        @pl.when((i == 0) | (be_ref[i] != prev))
        def _():
            w1s_ref[...] = w1_ref[0].astype(jnp.bfloat16)
            w2s_ref[...] = w2_ref[0].astype(jnp.bfloat16)

        e = be_ref[i]
        xb = xs_ref[...].astype(jnp.bfloat16)
        hh = _H // 2
        yb = b2_ref[pl.ds(e, 1), :] * 1.0
        b1e = b1_ref[pl.ds(e, 1), :]
        for p in range(2):
            w1h = w1s_ref[:, pl.ds(p * hh, hh)]
            b1h = lax.slice(b1e, (0, p * hh), (1, (p + 1) * hh))
            h = jnp.dot(xb, w1h, preferred_element_type=jnp.float32) + b1h
            h = 0.5 * h * (1.0 + lax.erf(h * 0.7071067811865476))
            yb = yb + jnp.dot(h.astype(jnp.bfloat16),
                              w2s_ref[pl.ds(p * hh, hh), :],
                              preferred_element_type=jnp.float32)
        ys_ref[...] = yb * ws_ref[:, 0:1]


def _ffn(be, bv, xs, ws, W1, b1, W2, b2):
    clamp = lambda i, be, bv: jnp.where(bv[i] == 1, i, 0)
    grid_spec = pltpu.PrefetchScalarGridSpec(
        num_scalar_prefetch=2,
        grid=(_NB,),
        in_specs=[
            pl.BlockSpec((_BS, _D), lambda i, be, bv: (clamp(i, be, bv), 0)),
            pl.BlockSpec((_BS, _WW), lambda i, be, bv: (clamp(i, be, bv), 0)),
            pl.BlockSpec((1, _D, _H), lambda i, be, bv: (be[i], 0, 0)),
            pl.BlockSpec((_E, _H), lambda i, be, bv: (0, 0)),
            pl.BlockSpec((1, _H, _D), lambda i, be, bv: (be[i], 0, 0)),
            pl.BlockSpec((_E, _D), lambda i, be, bv: (0, 0)),
        ],
        out_specs=pl.BlockSpec((_BS, _D), lambda i, be, bv: (i, 0)),
        scratch_shapes=[pltpu.VMEM((_D, _H), jnp.bfloat16),
                        pltpu.VMEM((_H, _D), jnp.bfloat16)],
    )
    return pl.pallas_call(
        _ffn_kernel,
        grid_spec=grid_spec,
        out_shape=jax.ShapeDtypeStruct((_SPAD, _D), jnp.float32),
        compiler_params=pltpu.CompilerParams(
            dimension_semantics=("arbitrary",)),
    )(be, bv, xs, ws, W1, b1, W2, b2)


def _gather_combine(ys, pos0, pos1):
    mesh = plsc.VectorSubcoreMesh(core_axis_name="c", subcore_axis_name="s")

    @functools.partial(
        pl.kernel, mesh=mesh,
        out_type=jax.ShapeDtypeStruct((_T, _D), jnp.float32),
        scratch_types=[pltpu.VMEM((_RPW,), jnp.int32),
                       pltpu.VMEM((_RPW,), jnp.int32),
                       pltpu.VMEM((_RPW // 2, _D), jnp.float32),
                       pltpu.VMEM((_RPW // 2, _D), jnp.float32),
                       pltpu.VMEM((_RPW // 2, _D), jnp.float32),
                       pltpu.VMEM((_RPW // 2, _D), jnp.float32),
                       pltpu.SemaphoreType.DMA,
                       pltpu.SemaphoreType.DMA,
                       pltpu.SemaphoreType.DMA,
                       pltpu.SemaphoreType.DMA],
    )
    def k(ys_hbm, p0_hbm, p1_hbm, y_hbm, idx0_v, idx1_v, a0_v, b0_v,
          a1_v, b1_v, s0, s1, s2, s3):
        wid = lax.axis_index("s") * _NC + lax.axis_index("c")
        base = wid * _RPW
        hp = _RPW // 2
        pltpu.sync_copy(p0_hbm.at[pl.ds(base, _RPW)], idx0_v)
        pltpu.sync_copy(p1_hbm.at[pl.ds(base, _RPW)], idx1_v)
        c0 = pltpu.async_copy(ys_hbm.at[idx0_v.at[pl.ds(0, hp)]], a0_v, s0)
        c1 = pltpu.async_copy(ys_hbm.at[idx1_v.at[pl.ds(0, hp)]], b0_v, s1)
        c2 = pltpu.async_copy(ys_hbm.at[idx0_v.at[pl.ds(hp, hp)]], a1_v, s2)
        c3 = pltpu.async_copy(ys_hbm.at[idx1_v.at[pl.ds(hp, hp)]], b1_v, s3)

        def add_store(a_v, b_v, off):
            @pl.loop(0, hp)
            def _(r):
                @pl.loop(0, _D, step=16)
                def _(cc):
                    slc = (pl.ds(r, 1), pl.ds(cc, 16))
                    a_v.at[*slc][...] = a_v.at[*slc][...] + b_v.at[*slc][...]
            pltpu.sync_copy(a_v, y_hbm.at[pl.ds(base + off, hp)])

        c0.wait()
        c1.wait()
        add_store(a0_v, b0_v, 0)
        c2.wait()
        c3.wait()
        add_store(a1_v, b1_v, hp)

    return k(ys, pos0, pos1)


def kernel(x, ln_gamma, ln_beta, router_W, router_b, W1, b1, W2, b2):
    xn, pos0, pos1, wp0, wp1, be2, bv2 = _route(
        x, ln_gamma, ln_beta, router_W, router_b)
    xs, ws = _scatter_rows(xn, wp0, wp1, pos0, pos1)
    ys = _ffn(be2, bv2, xs, ws, W1, b1, W2, b2)
    return _gather_combine(ys, pos0, pos1)
